# Initial kernel scaffold; baseline (speedup 1.0000x reference)
import jax
import jax.numpy as jnp
from jax.experimental import pallas as pl


def kernel(sphere, shape, features, w1, b1, g1, be1, w2, b2, g2, be2):
    raise NotImplementedError("write your pallas kernel here")


# speedup vs baseline: 1.0940x; 1.0940x over previous
import functools

import jax
import jax.numpy as jnp
from jax import lax
from jax.experimental import pallas as pl
from jax.experimental.pallas import tpu as pltpu
from jax.experimental.pallas import tpu_sc as plsc

_B, _N, _F, _G = 4, 2048, 64, 64
_EPS = 1e-5
_PPT = _N // 8
_ROW_CH = _F // 16
_SEG = _G * _F // 8
_NEG = -3.4028235e38


def _tc_body(sph_ref, shp_ref, w1_ref, b1_ref, g1_ref, be1_ref,
             w2_ref, b2_ref, g2_ref, be2_ref,
             soft_ref, wxyz_ref, grp_ref, cnt_ref, wfold_ref):
    sph = sph_ref[...].reshape(_B * _N, 3)
    h = lax.dot_general(sph, w1_ref[...], (((1,), (1,)), ((), ())),
                        preferred_element_type=jnp.float32)
    h = h + b1_ref[...][None, :]
    mean1 = jnp.mean(h, axis=0, keepdims=True)
    var1 = jnp.mean(h * h, axis=0, keepdims=True) - mean1 * mean1
    h = g1_ref[...][None, :] * (h - mean1) * lax.rsqrt(var1 + _EPS) \
        + be1_ref[...][None, :]
    h = jnp.maximum(h, 0.0)
    spm_t = lax.dot_general(w2_ref[...], h, (((1,), (1,)), ((), ())),
                            preferred_element_type=jnp.float32)
    spm_t = spm_t + b2_ref[...][:, None]
    mean2 = jnp.mean(spm_t, axis=1, keepdims=True)
    var2 = jnp.mean(spm_t * spm_t, axis=1, keepdims=True) - mean2 * mean2
    spm_t = g2_ref[...][:, None] * (spm_t - mean2) * lax.rsqrt(var2 + _EPS) \
        + be2_ref[...][:, None]
    for b in range(_B):
        z = spm_t[:, b * _N:(b + 1) * _N]
        m = jnp.max(z, axis=1, keepdims=True)
        e = jnp.exp(z - m)
        soft_b = e / jnp.sum(e, axis=1, keepdims=True)
        soft_ref[b] = soft_b
        g = jnp.argmax(z, axis=0).astype(jnp.int32)
        grp_ref[b] = g
        onehot = (lax.broadcasted_iota(jnp.int32, (_G, _N), 0) == g[None, :])
        cnt_ref[b] = jnp.sum(onehot.astype(jnp.int32), axis=1)
        wxyz_ref[b] = lax.dot_general(soft_b, sph_ref[b],
                                      (((1,), (0,)), ((), ())),
                                      preferred_element_type=jnp.float32)
        wfold_ref[b] = lax.dot_general(soft_b, shp_ref[b],
                                       (((1,), (0,)), ((), ())),
                                       preferred_element_type=jnp.float32)


def _tc_forward(sphere, shape, w1, b1, g1, be1, w2, b2, g2, be2):
    return pl.pallas_call(
        _tc_body,
        out_shape=[
            jax.ShapeDtypeStruct((_B, _G, _N), jnp.float32),
            jax.ShapeDtypeStruct((_B, _G, 3), jnp.float32),
            jax.ShapeDtypeStruct((_B, _N), jnp.int32),
            jax.ShapeDtypeStruct((_B, _G), jnp.int32),
            jax.ShapeDtypeStruct((_B, _G, 3), jnp.float32),
        ],
    )(sphere, shape, w1, b1, g1, be1, w2, b2, g2, be2)


def _sc_body(feat_hbm, grp_hbm, cnt_hbm, gf_hbm, sf_hbm,
             feat_v, grp_v, tbl_v, tmp_v, acc_v, cnt_v, slab_sh, gf_sh):
    c = lax.axis_index("c")
    s = lax.axis_index("s")
    lb = s // 8
    b = c * 2 + lb
    t8 = s % 8
    pt_base = b * _N + t8 * _PPT

    pltpu.sync_copy(feat_hbm.at[pl.ds(pt_base * _F, _PPT * _F)], feat_v)
    pltpu.sync_copy(grp_hbm.at[pl.ds(pt_base, _PPT)], grp_v)
    pltpu.sync_copy(cnt_hbm.at[pl.ds(b * _G, _G)], cnt_v.at[pl.ds(0, _G)])

    neg = jnp.full((16,), _NEG, jnp.float32)

    def init_body(i, carry):
        tbl_v[pl.ds(i * 16, 16)] = neg
        return carry

    lax.fori_loop(0, _G * _F // 16, init_body, 0)

    def scat_body(i, carry):
        gvec = grp_v[pl.ds(i * 16, 16)]
        for k in range(16):
            base = gvec[k] * _F
            fb = (i * 16 + k) * _F
            for j in range(_ROW_CH):
                row = feat_v[pl.ds(fb + j * 16, 16)]
                cur = tbl_v[pl.ds(base + j * 16, 16)]
                tbl_v[pl.ds(base + j * 16, 16)] = jnp.maximum(cur, row)
        return carry

    lax.fori_loop(0, _PPT // 16, scat_body, 0)

    pltpu.sync_copy(tbl_v, slab_sh.at[s])
    plsc.subcore_barrier()

    seg = t8 * _SEG
    pltpu.sync_copy(slab_sh.at[lb * 8, pl.ds(seg, _SEG)], acc_v)
    for t in range(1, 8):
        pltpu.sync_copy(slab_sh.at[lb * 8 + t, pl.ds(seg, _SEG)], tmp_v)
        for v in range(_SEG // 16):
            sl = pl.ds(v * 16, 16)
            acc_v[sl] = jnp.maximum(acc_v[sl], tmp_v[sl])
    cvec = cnt_v[pl.ds(t8 * 8, 16)]
    for r in range(8):
        floor = jnp.where(cvec[r] == _N, _NEG, 0.0)
        fv = jnp.full((16,), floor)
        for j in range(_ROW_CH):
            sl = pl.ds(r * _F + j * 16, 16)
            acc_v[sl] = jnp.maximum(acc_v[sl], fv)

    pltpu.sync_copy(acc_v, gf_hbm.at[pl.ds(b * _G * _F + seg, _SEG)])
    pltpu.sync_copy(acc_v, gf_sh.at[lb, pl.ds(seg, _SEG)])
    plsc.subcore_barrier()
    pltpu.sync_copy(gf_sh.at[lb], tbl_v)

    def gath_body(i, carry):
        gvec = grp_v[pl.ds(i * 16, 16)]
        for k in range(16):
            base = gvec[k] * _F
            fb = (i * 16 + k) * _F
            for j in range(_ROW_CH):
                feat_v[pl.ds(fb + j * 16, 16)] = tbl_v[pl.ds(base + j * 16, 16)]
        return carry

    lax.fori_loop(0, _PPT // 16, gath_body, 0)
    pltpu.sync_copy(feat_v, sf_hbm.at[pl.ds(pt_base * _F, _PPT * _F)])


@functools.cache
def _sc_combine():
    return pl.kernel(
        _sc_body,
        mesh=plsc.VectorSubcoreMesh(core_axis_name="c", subcore_axis_name="s",
                                    num_cores=2),
        out_type=[
            jax.ShapeDtypeStruct((_B * _G * _F,), jnp.float32),
            jax.ShapeDtypeStruct((_B * _N * _F,), jnp.float32),
        ],
        scratch_types=[
            pltpu.VMEM((_PPT * _F,), jnp.float32),
            pltpu.VMEM((_PPT,), jnp.int32),
            pltpu.VMEM((_G * _F,), jnp.float32),
            pltpu.VMEM((_SEG,), jnp.float32),
            pltpu.VMEM((_SEG,), jnp.float32),
            pltpu.VMEM((_G + 16,), jnp.int32),
            pltpu.VMEM_SHARED((16, _G * _F), jnp.float32),
            pltpu.VMEM_SHARED((2, _G * _F), jnp.float32),
        ],
    )


def kernel(sphere, shape, features, w1, b1, g1, be1, w2, b2, g2, be2):
    soft, weighted_xyz, groups, counts, weighted_folded = _tc_forward(
        sphere, shape, w1, b1, g1, be1, w2, b2, g2, be2)
    gf_flat, sf_flat = _sc_combine()(
        features.reshape(-1), groups.reshape(-1), counts.reshape(-1))
    group_features = gf_flat.reshape(_B, _G, _F)
    scattered_features = sf_flat.reshape(_B, _N, _F)
    return (soft, weighted_xyz, groups, group_features,
            scattered_features, weighted_folded)


# speedup vs baseline: 1.2264x; 1.1210x over previous
import functools

import jax
import jax.numpy as jnp
from jax import lax
from jax.experimental import pallas as pl
from jax.experimental.pallas import tpu as pltpu
from jax.experimental.pallas import tpu_sc as plsc

_B, _N, _F, _G = 4, 2048, 64, 64
_EPS = 1e-5
_PPT = _N // 8
_ROW_CH = _F // 16
_GPT = _G // 8
_NEG = -3.4028235e38


def _tc_body(sph_ref, shp_ref, w1_ref, b1_ref, g1_ref, be1_ref,
             w2_ref, b2_ref, g2_ref, be2_ref,
             soft_ref, wxyz_ref, grp_ref, cnt_ref, wfold_ref):
    sph = sph_ref[...].reshape(_B * _N, 3)
    h = lax.dot_general(sph, w1_ref[...], (((1,), (1,)), ((), ())),
                        preferred_element_type=jnp.float32)
    h = h + b1_ref[...][None, :]
    mean1 = jnp.mean(h, axis=0, keepdims=True)
    var1 = jnp.mean(h * h, axis=0, keepdims=True) - mean1 * mean1
    a1 = g1_ref[...][None, :] * lax.rsqrt(var1 + _EPS)
    c1 = be1_ref[...][None, :] - mean1 * a1
    h = jnp.maximum(h * a1 + c1, 0.0)
    spm_t = lax.dot_general(w2_ref[...], h, (((1,), (1,)), ((), ())),
                            preferred_element_type=jnp.float32)
    spm_t = spm_t + b2_ref[...][:, None]
    mean2 = jnp.mean(spm_t, axis=1, keepdims=True)
    var2 = jnp.mean(spm_t * spm_t, axis=1, keepdims=True) - mean2 * mean2
    a2 = g2_ref[...][:, None] * lax.rsqrt(var2 + _EPS)
    c2 = be2_ref[...][:, None] - mean2 * a2
    spm_t = spm_t * a2 + c2
    for b in range(_B):
        z = spm_t[:, b * _N:(b + 1) * _N]
        m = jnp.max(z, axis=1, keepdims=True)
        e = jnp.exp(z - m)
        soft_b = e * (1.0 / jnp.sum(e, axis=1, keepdims=True))
        soft_ref[b] = soft_b
        g = jnp.argmax(z, axis=0).astype(jnp.int32)
        grp_ref[b] = g
        onehot = (lax.broadcasted_iota(jnp.int32, (_G, _N), 0) == g[None, :])
        cnt_ref[pl.ds(b * _G, _G)] = jnp.sum(onehot.astype(jnp.int32), axis=1)
        wxyz_ref[b] = lax.dot_general(soft_b, sph_ref[b],
                                      (((1,), (0,)), ((), ())),
                                      preferred_element_type=jnp.float32)
        wfold_ref[b] = lax.dot_general(soft_b, shp_ref[b],
                                       (((1,), (0,)), ((), ())),
                                       preferred_element_type=jnp.float32)


def _tc_forward(sphere, shape, w1, b1, g1, be1, w2, b2, g2, be2):
    return pl.pallas_call(
        _tc_body,
        out_shape=[
            jax.ShapeDtypeStruct((_B, _G, _N), jnp.float32),
            jax.ShapeDtypeStruct((_B, _G, 3), jnp.float32),
            jax.ShapeDtypeStruct((_B, _N), jnp.int32),
            jax.ShapeDtypeStruct((_B * _G,), jnp.int32),
            jax.ShapeDtypeStruct((_B, _G, 3), jnp.float32),
        ],
    )(sphere, shape, w1, b1, g1, be1, w2, b2, g2, be2)


def _sc_body(feat_hbm, grp_hbm, cnt_hbm, gf_hbm, sf_hbm, slab_hbm,
             feat_v, grp_v, tbl_v, tmp_v, acc_v, cnt_v, sem_f, sem_r):
    c = lax.axis_index("c")
    s = lax.axis_index("s")
    lb = s // 8
    b = c * 2 + lb
    t8 = s % 8
    n0 = t8 * _PPT

    feat_cp = pltpu.async_copy(feat_hbm.at[b, pl.ds(n0, _PPT)], feat_v, sem_f)
    pltpu.sync_copy(grp_hbm.at[b, pl.ds(n0, _PPT)], grp_v)
    pltpu.sync_copy(cnt_hbm.at[pl.ds(b * _G, _G)], cnt_v.at[pl.ds(0, _G)])

    neg = jnp.full((16,), _NEG, jnp.float32)

    def init_body(r, carry):
        for j in range(_ROW_CH):
            tbl_v[r, pl.ds(j * 16, 16)] = neg
        return carry

    lax.fori_loop(0, _G, init_body, 0)
    feat_cp.wait()

    def scat_body(i, carry):
        gvec = grp_v[pl.ds(i * 16, 16)]
        for k in range(16):
            gid = gvec[k]
            pt = i * 16 + k
            for j in range(_ROW_CH):
                row = feat_v[pt, pl.ds(j * 16, 16)]
                cur = tbl_v[gid, pl.ds(j * 16, 16)]
                tbl_v[gid, pl.ds(j * 16, 16)] = jnp.maximum(cur, row)
        return carry

    lax.fori_loop(0, _PPT // 16, scat_body, 0)

    pltpu.sync_copy(tbl_v, slab_hbm.at[c * 16 + s])
    plsc.subcore_barrier()

    g0 = t8 * _GPT
    base = c * 16 + lb * 8
    copies = [pltpu.async_copy(
        slab_hbm.at[base + lax.rem(t8 + t, 8), pl.ds(g0, _GPT)],
        tmp_v.at[t], sem_r) for t in range(1, 8)]
    for cp in copies:
        cp.wait()
    for r in range(_GPT):
        for j in range(_ROW_CH):
            sl = pl.ds(j * 16, 16)
            m01 = jnp.maximum(tbl_v[g0 + r, sl], tmp_v[1, r, sl])
            m23 = jnp.maximum(tmp_v[2, r, sl], tmp_v[3, r, sl])
            m45 = jnp.maximum(tmp_v[4, r, sl], tmp_v[5, r, sl])
            m67 = jnp.maximum(tmp_v[6, r, sl], tmp_v[7, r, sl])
            acc_v[r, sl] = jnp.maximum(jnp.maximum(m01, m23),
                                       jnp.maximum(m45, m67))
    cvec = cnt_v[pl.ds(g0, 16)]
    for r in range(_GPT):
        floor = jnp.where(cvec[r] == _N, _NEG, 0.0)
        fv = jnp.full((16,), floor)
        for j in range(_ROW_CH):
            sl = pl.ds(j * 16, 16)
            acc_v[r, sl] = jnp.maximum(acc_v[r, sl], fv)

    pltpu.sync_copy(acc_v, gf_hbm.at[b, pl.ds(g0, _GPT)])
    plsc.subcore_barrier()
    pltpu.sync_copy(gf_hbm.at[b], tbl_v)

    def gath_body(i, carry):
        gvec = grp_v[pl.ds(i * 16, 16)]
        for k in range(16):
            gid = gvec[k]
            pt = i * 16 + k
            for j in range(_ROW_CH):
                feat_v[pt, pl.ds(j * 16, 16)] = tbl_v[gid, pl.ds(j * 16, 16)]
        return carry

    lax.fori_loop(0, _PPT // 16, gath_body, 0)
    pltpu.sync_copy(feat_v, sf_hbm.at[b, pl.ds(n0, _PPT)])


@functools.cache
def _sc_combine():
    return pl.kernel(
        _sc_body,
        mesh=plsc.VectorSubcoreMesh(core_axis_name="c", subcore_axis_name="s",
                                    num_cores=2),
        out_type=[
            jax.ShapeDtypeStruct((_B, _G, _F), jnp.float32),
            jax.ShapeDtypeStruct((_B, _N, _F), jnp.float32),
            jax.ShapeDtypeStruct((32, _G, _F), jnp.float32),
        ],
        scratch_types=[
            pltpu.VMEM((_PPT, _F), jnp.float32),
            pltpu.VMEM((_PPT,), jnp.int32),
            pltpu.VMEM((_G, _F), jnp.float32),
            pltpu.VMEM((8, _GPT, _F), jnp.float32),
            pltpu.VMEM((_GPT, _F), jnp.float32),
            pltpu.VMEM((_G + 16,), jnp.int32),
            pltpu.SemaphoreType.DMA,
            pltpu.SemaphoreType.DMA,
        ],
    )


def kernel(sphere, shape, features, w1, b1, g1, be1, w2, b2, g2, be2):
    soft, weighted_xyz, groups, counts, weighted_folded = _tc_forward(
        sphere, shape, w1, b1, g1, be1, w2, b2, g2, be2)
    group_features, scattered_features, _ = _sc_combine()(
        features, groups, counts)
    return (soft, weighted_xyz, groups, group_features,
            scattered_features, weighted_folded)


# speedup vs baseline: 1.2271x; 1.0006x over previous
import functools

import jax
import jax.numpy as jnp
from jax import lax
from jax.experimental import pallas as pl
from jax.experimental.pallas import tpu as pltpu
from jax.experimental.pallas import tpu_sc as plsc

_B, _N, _F, _G = 4, 2048, 64, 64
_EPS = 1e-5
_PPT = _N // 8
_ROW_CH = _F // 16
_GPT = _G // 8
_NEG = -3.4028235e38


def _tc_body(sph_ref, shp_ref, w1_ref, b1_ref, g1_ref, be1_ref,
             w2_ref, b2_ref, g2_ref, be2_ref,
             soft_ref, wxyz_ref, grp_ref, cnt_ref, wfold_ref):
    nn = float(_B * _N)
    sph = sph_ref[...].reshape(_B * _N, 3)
    w1 = w1_ref[...]
    h = lax.dot_general(sph, w1, (((1,), (1,)), ((), ())),
                        preferred_element_type=jnp.float32)
    mean1 = jnp.mean(h, axis=0, keepdims=True)
    var1 = jnp.mean(h * h, axis=0, keepdims=True) - mean1 * mean1
    a1 = g1_ref[...][None, :] * lax.rsqrt(var1 + _EPS)
    c1 = be1_ref[...][None, :] - mean1 * a1
    h = jnp.maximum(h * a1 + c1, 0.0)
    w2 = w2_ref[...]
    spm_t = lax.dot_general(w2, h, (((1,), (1,)), ((), ())),
                            preferred_element_type=jnp.float32)
    mean2 = jnp.mean(spm_t, axis=1, keepdims=True)
    var2 = jnp.mean(spm_t * spm_t, axis=1, keepdims=True) - mean2 * mean2
    a2 = g2_ref[...][:, None] * lax.rsqrt(var2 + _EPS)
    c2 = be2_ref[...][:, None] - mean2 * a2
    spm_t = spm_t * a2 + c2
    for b in range(_B):
        z = spm_t[:, b * _N:(b + 1) * _N]
        m = jnp.max(z, axis=1, keepdims=True)
        e = jnp.exp(z - m)
        soft_b = e * (1.0 / jnp.sum(e, axis=1, keepdims=True))
        soft_ref[b] = soft_b
        g = jnp.argmax(z, axis=0).astype(jnp.int32)
        grp_ref[b] = g
        onehot = (lax.broadcasted_iota(jnp.int32, (_G, _N), 0) == g[None, :])
        cnt_ref[pl.ds(b * _G, _G)] = jnp.sum(onehot.astype(jnp.int32), axis=1)
        wxyz_ref[b] = lax.dot_general(soft_b, sph_ref[b],
                                      (((1,), (0,)), ((), ())),
                                      preferred_element_type=jnp.float32)
        wfold_ref[b] = lax.dot_general(soft_b, shp_ref[b],
                                       (((1,), (0,)), ((), ())),
                                       preferred_element_type=jnp.float32)


def _tc_forward(sphere, shape, w1, b1, g1, be1, w2, b2, g2, be2):
    return pl.pallas_call(
        _tc_body,
        out_shape=[
            jax.ShapeDtypeStruct((_B, _G, _N), jnp.float32),
            jax.ShapeDtypeStruct((_B, _G, 3), jnp.float32),
            jax.ShapeDtypeStruct((_B, _N), jnp.int32),
            jax.ShapeDtypeStruct((_B * _G,), jnp.int32),
            jax.ShapeDtypeStruct((_B, _G, 3), jnp.float32),
        ],
    )(sphere, shape, w1, b1, g1, be1, w2, b2, g2, be2)


def _sc_body(feat_hbm, grp_hbm, cnt_hbm, gf_hbm, sf_hbm, slab_hbm,
             feat_v, grp_v, tbl_v, tmp_v, acc_v, cnt_v, sem_f, sem_r):
    c = lax.axis_index("c")
    s = lax.axis_index("s")
    lb = s // 8
    b = c * 2 + lb
    t8 = s % 8
    n0 = t8 * _PPT

    feat_cp = pltpu.async_copy(feat_hbm.at[b, pl.ds(n0, _PPT)], feat_v, sem_f)
    grp_cp = pltpu.async_copy(grp_hbm.at[b, pl.ds(n0, _PPT)], grp_v, sem_r)
    cnt_cp = pltpu.async_copy(cnt_hbm.at[pl.ds(b * _G, _G)],
                              cnt_v.at[pl.ds(0, _G)], sem_f)

    neg = jnp.full((16,), _NEG, jnp.float32)

    def init_body(r, carry):
        for j in range(_ROW_CH):
            tbl_v[r, pl.ds(j * 16, 16)] = neg
        return carry

    lax.fori_loop(0, _G, init_body, 0)
    grp_cp.wait()
    feat_cp.wait()
    cnt_cp.wait()

    def scat_body(i, carry):
        gvec = grp_v[pl.ds(i * 16, 16)]
        for k in range(16):
            gid = gvec[k]
            pt = i * 16 + k
            for j in range(_ROW_CH):
                row = feat_v[pt, pl.ds(j * 16, 16)]
                cur = tbl_v[gid, pl.ds(j * 16, 16)]
                tbl_v[gid, pl.ds(j * 16, 16)] = jnp.maximum(cur, row)
        return carry

    lax.fori_loop(0, _PPT // 16, scat_body, 0)

    pltpu.sync_copy(tbl_v, slab_hbm.at[c * 16 + s])
    plsc.subcore_barrier()

    g0 = t8 * _GPT
    base = c * 16 + lb * 8
    copies = [pltpu.async_copy(
        slab_hbm.at[base + lax.rem(t8 + t, 8), pl.ds(g0, _GPT)],
        tmp_v.at[t], sem_r) for t in range(1, 8)]
    for cp in copies:
        cp.wait()
    for r in range(_GPT):
        for j in range(_ROW_CH):
            sl = pl.ds(j * 16, 16)
            m01 = jnp.maximum(tbl_v[g0 + r, sl], tmp_v[1, r, sl])
            m23 = jnp.maximum(tmp_v[2, r, sl], tmp_v[3, r, sl])
            m45 = jnp.maximum(tmp_v[4, r, sl], tmp_v[5, r, sl])
            m67 = jnp.maximum(tmp_v[6, r, sl], tmp_v[7, r, sl])
            acc_v[r, sl] = jnp.maximum(jnp.maximum(m01, m23),
                                       jnp.maximum(m45, m67))
    cvec = cnt_v[pl.ds(g0, 16)]
    for r in range(_GPT):
        floor = jnp.where(cvec[r] == _N, _NEG, 0.0)
        fv = jnp.full((16,), floor)
        for j in range(_ROW_CH):
            sl = pl.ds(j * 16, 16)
            acc_v[r, sl] = jnp.maximum(acc_v[r, sl], fv)

    pltpu.sync_copy(acc_v, gf_hbm.at[b, pl.ds(g0, _GPT)])
    plsc.subcore_barrier()
    pltpu.sync_copy(gf_hbm.at[b], tbl_v)

    def gath_body(i, carry):
        gvec = grp_v[pl.ds(i * 16, 16)]
        for k in range(16):
            gid = gvec[k]
            pt = i * 16 + k
            for j in range(_ROW_CH):
                feat_v[pt, pl.ds(j * 16, 16)] = tbl_v[gid, pl.ds(j * 16, 16)]
        return carry

    lax.fori_loop(0, _PPT // 16, gath_body, 0)
    pltpu.sync_copy(feat_v, sf_hbm.at[b, pl.ds(n0, _PPT)])


@functools.cache
def _sc_combine():
    return pl.kernel(
        _sc_body,
        mesh=plsc.VectorSubcoreMesh(core_axis_name="c", subcore_axis_name="s",
                                    num_cores=2),
        out_type=[
            jax.ShapeDtypeStruct((_B, _G, _F), jnp.float32),
            jax.ShapeDtypeStruct((_B, _N, _F), jnp.float32),
            jax.ShapeDtypeStruct((32, _G, _F), jnp.float32),
        ],
        scratch_types=[
            pltpu.VMEM((_PPT, _F), jnp.float32),
            pltpu.VMEM((_PPT,), jnp.int32),
            pltpu.VMEM((_G, _F), jnp.float32),
            pltpu.VMEM((8, _GPT, _F), jnp.float32),
            pltpu.VMEM((_GPT, _F), jnp.float32),
            pltpu.VMEM((_G + 16,), jnp.int32),
            pltpu.SemaphoreType.DMA,
            pltpu.SemaphoreType.DMA,
        ],
    )


def kernel(sphere, shape, features, w1, b1, g1, be1, w2, b2, g2, be2):
    soft, weighted_xyz, groups, counts, weighted_folded = _tc_forward(
        sphere, shape, w1, b1, g1, be1, w2, b2, g2, be2)
    group_features, scattered_features, _ = _sc_combine()(
        features, groups, counts)
    return (soft, weighted_xyz, groups, group_features,
            scattered_features, weighted_folded)
